# Initial kernel scaffold; baseline (speedup 1.0000x reference)
import jax
import jax.numpy as jnp
from jax.experimental import pallas as pl


def kernel(x, edge_index, W1, b1, W2, b2, W3, b3, W4, b4, W5, b5):
    raise NotImplementedError("write your pallas kernel here")


# speedup vs baseline: 5.3868x; 5.3868x over previous
import functools

import jax
import jax.numpy as jnp
from jax import lax
from jax.experimental import pallas as pl
from jax.experimental.pallas import tpu as pltpu
from jax.experimental.pallas import tpu_sc as plsc

N = 10000
E = 320000
D = 128
H = 64

_NC = 2
_NS = 16
_NW = _NC * _NS
_EPW = E // _NW
_CHUNK = 80
_NSTEPS = _EPW // _CHUNK
_RPT = 624
_TAIL = N - _RPT * _NS

_R = 2000
_GRID = N // _R


def _sc_body(g_hbm, src_hbm, dst_hbm, zero_hbm, out_hbm, sidx, didx, rows,
             acc, sem):
    cid = lax.axis_index("c")
    sid = lax.axis_index("s")
    wid = cid * _NS + sid

    pltpu.sync_copy(zero_hbm.at[pl.ds(sid * _RPT, _RPT)],
                    acc.at[pl.ds(sid * _RPT, _RPT)])

    @pl.when(sid == _NS - 1)
    def _():
        pltpu.sync_copy(zero_hbm.at[pl.ds(_RPT * _NS, _TAIL)],
                        acc.at[pl.ds(_RPT * _NS, _TAIL)])

    plsc.subcore_barrier()

    base = wid * _EPW

    def step(i, carry):
        off = base + i * _CHUNK
        pltpu.sync_copy(src_hbm.at[pl.ds(off, _CHUNK)], sidx)
        pltpu.sync_copy(dst_hbm.at[pl.ds(off, _CHUNK)], didx)
        pltpu.async_copy(g_hbm.at[sidx], rows, sem).wait()
        pltpu.sync_copy(rows, acc.at[didx], add=True)
        return carry

    lax.fori_loop(0, _NSTEPS, step, 0)

    plsc.subcore_barrier()
    pltpu.sync_copy(acc.at[pl.ds(sid * _RPT, _RPT)],
                    out_hbm.at[cid, pl.ds(sid * _RPT, _RPT)])

    @pl.when(sid == _NS - 1)
    def _():
        pltpu.sync_copy(acc.at[pl.ds(_RPT * _NS, _TAIL)],
                        out_hbm.at[cid, pl.ds(_RPT * _NS, _TAIL)])


@functools.cache
def _get_sc_segsum():
    return functools.partial(
        pl.kernel,
        out_type=jax.ShapeDtypeStruct((_NC, N, H), jnp.float32),
        mesh=plsc.VectorSubcoreMesh(core_axis_name="c", subcore_axis_name="s",
                                    num_cores=_NC, num_subcores=_NS),
        scratch_types=[
            pltpu.VMEM((_CHUNK,), jnp.int32),
            pltpu.VMEM((_CHUNK,), jnp.int32),
            pltpu.VMEM((_CHUNK, H), jnp.float32),
            pltpu.VMEM_SHARED((N, H), jnp.float32),
            pltpu.SemaphoreType.DMA,
        ],
        compiler_params=pltpu.CompilerParams(use_tc_tiling_on_sc=False),
    )(_sc_body)


def _sc_segsum(g, src, dst, zeros):
    return _get_sc_segsum()(g, src, dst, zeros)


def _pre_body(x_ref, w1_ref, b1_ref, w3_ref, h_ref, g_ref):
    h = jnp.dot(x_ref[...], w1_ref[...], preferred_element_type=jnp.float32)
    h = jnp.maximum(h + b1_ref[...], 0.0)
    hn = h * lax.rsqrt(jnp.sum(h * h, axis=1, keepdims=True))
    h_ref[...] = hn
    g_ref[...] = jnp.dot(hn, w3_ref[...], preferred_element_type=jnp.float32)


def _tc_pre(x, W1, b1, W3):
    return pl.pallas_call(
        _pre_body,
        grid=(_GRID,),
        in_specs=[
            pl.BlockSpec((_R, D), lambda i: (i, 0)),
            pl.BlockSpec((D, D), lambda i: (0, 0)),
            pl.BlockSpec((1, D), lambda i: (0, 0)),
            pl.BlockSpec((D, H), lambda i: (0, 0)),
        ],
        out_specs=[
            pl.BlockSpec((_R, D), lambda i: (i, 0)),
            pl.BlockSpec((_R, H), lambda i: (i, 0)),
        ],
        out_shape=[
            jax.ShapeDtypeStruct((N, D), jnp.float32),
            jax.ShapeDtypeStruct((N, H), jnp.float32),
        ],
    )(x, W1, b1.reshape(1, D), W3)


def _combine(h, p0, p1, w2, b2, b3):
    l = jnp.dot(h, w2, preferred_element_type=jnp.float32) + b2
    l = jnp.maximum(l, 0.0)
    r = jnp.maximum(p0 + p1 + b3, 0.0)
    ss = jnp.sum(l * l, axis=1, keepdims=True) + jnp.sum(r * r, axis=1,
                                                         keepdims=True)
    n = lax.rsqrt(ss)
    return l * n, r * n


def _mid_body(h_ref, p0_ref, p1_ref, w2_ref, b2_ref, b3_ref, w3_ref,
              h_out_ref, g_ref):
    ln, rn = _combine(h_ref[...], p0_ref[0], p1_ref[0], w2_ref[...],
                      b2_ref[...], b3_ref[...])
    h_out_ref[:, :H] = ln
    h_out_ref[:, H:] = rn
    g_ref[...] = (
        jnp.dot(ln, w3_ref[:H, :], preferred_element_type=jnp.float32)
        + jnp.dot(rn, w3_ref[H:, :], preferred_element_type=jnp.float32))


def _tc_mid(h, parts, W2, b2, b3, W3):
    return pl.pallas_call(
        _mid_body,
        grid=(_GRID,),
        in_specs=[
            pl.BlockSpec((_R, D), lambda i: (i, 0)),
            pl.BlockSpec((1, _R, H), lambda i: (0, i, 0)),
            pl.BlockSpec((1, _R, H), lambda i: (1, i, 0)),
            pl.BlockSpec((D, H), lambda i: (0, 0)),
            pl.BlockSpec((1, H), lambda i: (0, 0)),
            pl.BlockSpec((1, H), lambda i: (0, 0)),
            pl.BlockSpec((D, H), lambda i: (0, 0)),
        ],
        out_specs=[
            pl.BlockSpec((_R, D), lambda i: (i, 0)),
            pl.BlockSpec((_R, H), lambda i: (i, 0)),
        ],
        out_shape=[
            jax.ShapeDtypeStruct((N, D), jnp.float32),
            jax.ShapeDtypeStruct((N, H), jnp.float32),
        ],
    )(h, parts, parts, W2, b2.reshape(1, H), b3.reshape(1, H), W3)


def _post_body(h_ref, p0_ref, p1_ref, w2_ref, b2_ref, b3_ref, w4_ref,
               b4_ref, w5_ref, b5_ref, out_ref):
    ln, rn = _combine(h_ref[...], p0_ref[0], p1_ref[0], w2_ref[...],
                      b2_ref[...], b3_ref[...])
    h4 = (jnp.dot(ln, w4_ref[:H, :], preferred_element_type=jnp.float32)
          + jnp.dot(rn, w4_ref[H:, :], preferred_element_type=jnp.float32)
          + b4_ref[...])
    h4 = jnp.maximum(h4, 0.0)
    out_ref[...] = (jnp.sum(h4 * w5_ref[...], axis=1, keepdims=True)
                    + b5_ref[...])


def _tc_post(h, parts, W2, b2, b3, W4, b4, W5, b5):
    return pl.pallas_call(
        _post_body,
        grid=(_GRID,),
        in_specs=[
            pl.BlockSpec((_R, D), lambda i: (i, 0)),
            pl.BlockSpec((1, _R, H), lambda i: (0, i, 0)),
            pl.BlockSpec((1, _R, H), lambda i: (1, i, 0)),
            pl.BlockSpec((D, H), lambda i: (0, 0)),
            pl.BlockSpec((1, H), lambda i: (0, 0)),
            pl.BlockSpec((1, H), lambda i: (0, 0)),
            pl.BlockSpec((D, D), lambda i: (0, 0)),
            pl.BlockSpec((1, D), lambda i: (0, 0)),
            pl.BlockSpec((1, D), lambda i: (0, 0)),
            pl.BlockSpec((1, 1), lambda i: (0, 0)),
        ],
        out_specs=pl.BlockSpec((_R, 1), lambda i: (i, 0)),
        out_shape=jax.ShapeDtypeStruct((N, 1), jnp.float32),
    )(h, parts, parts, W2, b2.reshape(1, H), b3.reshape(1, H), W4,
      b4.reshape(1, D), W5.reshape(1, D), b5.reshape(1, 1))


def kernel(x, edge_index, W1, b1, W2, b2, W3, b3, W4, b4, W5, b5):
    src = edge_index[0].astype(jnp.int32)
    dst = edge_index[1].astype(jnp.int32)
    zeros = jnp.zeros((N, H), jnp.float32)

    h, g = _tc_pre(x, W1, b1, W3)
    parts = _sc_segsum(g, src, dst, zeros)
    h, g = _tc_mid(h, parts, W2, b2, b3, W3)
    parts = _sc_segsum(g, src, dst, zeros)
    return _tc_post(h, parts, W2, b2, b3, W4, b4, W5, b5)


# speedup vs baseline: 9.8655x; 1.8314x over previous
import functools

import jax
import jax.numpy as jnp
from jax import lax
from jax.experimental import pallas as pl
from jax.experimental.pallas import tpu as pltpu
from jax.experimental.pallas import tpu_sc as plsc

N = 10000
E = 320000
D = 128
H = 64

_NC = 2
_NS = 16
_NW = _NC * _NS
_EPW = E // _NW
_CHUNK = 80
_NSTEPS = _EPW // _CHUNK
_RPT = 624
_TAIL = N - _RPT * _NS

_R = 2000
_GRID = N // _R


def _sc_body(g_hbm, src_hbm, dst_hbm, zero_hbm, out_hbm, sidx, didx, rows,
             acc, sem):
    cid = lax.axis_index("c")
    sid = lax.axis_index("s")
    wid = cid * _NS + sid

    s_cp = pltpu.async_copy(src_hbm.at[wid], sidx, sem)
    d_cp = pltpu.async_copy(dst_hbm.at[wid], didx, sem)

    pltpu.sync_copy(zero_hbm.at[pl.ds(sid * _RPT, _RPT)],
                    acc.at[pl.ds(sid * _RPT, _RPT)])

    @pl.when(sid == _NS - 1)
    def _():
        pltpu.sync_copy(zero_hbm.at[pl.ds(_RPT * _NS, _TAIL)],
                        acc.at[pl.ds(_RPT * _NS, _TAIL)])

    s_cp.wait()
    d_cp.wait()
    plsc.subcore_barrier()

    def gather(i, b):
        return pltpu.async_copy(g_hbm.at[sidx.at[i]], rows.at[b], sem)

    gather(0, 0).wait()

    def step(i, carry):
        b = lax.rem(i, 2)
        nxt = gather(lax.rem(i + 1, _NSTEPS), 1 - b)
        pltpu.sync_copy(rows.at[b], acc.at[didx.at[i]], add=True)
        nxt.wait()
        return carry

    lax.fori_loop(0, _NSTEPS, step, 0)

    plsc.subcore_barrier()
    pltpu.sync_copy(acc.at[pl.ds(sid * _RPT, _RPT)],
                    out_hbm.at[cid, pl.ds(sid * _RPT, _RPT)])

    @pl.when(sid == _NS - 1)
    def _():
        pltpu.sync_copy(acc.at[pl.ds(_RPT * _NS, _TAIL)],
                        out_hbm.at[cid, pl.ds(_RPT * _NS, _TAIL)])


@functools.cache
def _get_sc_segsum():
    return functools.partial(
        pl.kernel,
        out_type=jax.ShapeDtypeStruct((_NC, N, H), jnp.float32),
        mesh=plsc.VectorSubcoreMesh(core_axis_name="c", subcore_axis_name="s",
                                    num_cores=_NC, num_subcores=_NS),
        scratch_types=[
            pltpu.VMEM((_NSTEPS, _CHUNK), jnp.int32),
            pltpu.VMEM((_NSTEPS, _CHUNK), jnp.int32),
            pltpu.VMEM((2, _CHUNK, H), jnp.float32),
            pltpu.VMEM_SHARED((N, H), jnp.float32),
            pltpu.SemaphoreType.DMA,
        ],
        compiler_params=pltpu.CompilerParams(use_tc_tiling_on_sc=False),
    )(_sc_body)


def _sc_segsum(g, src, dst, zeros):
    return _get_sc_segsum()(g, src, dst, zeros)


def _pre_body(x_ref, w1_ref, b1_ref, w3_ref, h_ref, g_ref):
    h = jnp.dot(x_ref[...], w1_ref[...], preferred_element_type=jnp.float32)
    h = jnp.maximum(h + b1_ref[...], 0.0)
    hn = h * lax.rsqrt(jnp.sum(h * h, axis=1, keepdims=True))
    h_ref[...] = hn
    g_ref[...] = jnp.dot(hn, w3_ref[...], preferred_element_type=jnp.float32)


def _tc_pre(x, W1, b1, W3):
    return pl.pallas_call(
        _pre_body,
        grid=(_GRID,),
        in_specs=[
            pl.BlockSpec((_R, D), lambda i: (i, 0)),
            pl.BlockSpec((D, D), lambda i: (0, 0)),
            pl.BlockSpec((1, D), lambda i: (0, 0)),
            pl.BlockSpec((D, H), lambda i: (0, 0)),
        ],
        out_specs=[
            pl.BlockSpec((_R, D), lambda i: (i, 0)),
            pl.BlockSpec((_R, H), lambda i: (i, 0)),
        ],
        out_shape=[
            jax.ShapeDtypeStruct((N, D), jnp.float32),
            jax.ShapeDtypeStruct((N, H), jnp.float32),
        ],
    )(x, W1, b1.reshape(1, D), W3)


def _combine(h, p0, p1, w2, b2, b3):
    l = jnp.dot(h, w2, preferred_element_type=jnp.float32) + b2
    l = jnp.maximum(l, 0.0)
    r = jnp.maximum(p0 + p1 + b3, 0.0)
    ss = jnp.sum(l * l, axis=1, keepdims=True) + jnp.sum(r * r, axis=1,
                                                         keepdims=True)
    n = lax.rsqrt(ss)
    return l * n, r * n


def _mid_body(h_ref, p0_ref, p1_ref, w2_ref, b2_ref, b3_ref, w3_ref,
              h_out_ref, g_ref):
    ln, rn = _combine(h_ref[...], p0_ref[0], p1_ref[0], w2_ref[...],
                      b2_ref[...], b3_ref[...])
    h_out_ref[:, :H] = ln
    h_out_ref[:, H:] = rn
    g_ref[...] = (
        jnp.dot(ln, w3_ref[:H, :], preferred_element_type=jnp.float32)
        + jnp.dot(rn, w3_ref[H:, :], preferred_element_type=jnp.float32))


def _tc_mid(h, parts, W2, b2, b3, W3):
    return pl.pallas_call(
        _mid_body,
        grid=(_GRID,),
        in_specs=[
            pl.BlockSpec((_R, D), lambda i: (i, 0)),
            pl.BlockSpec((1, _R, H), lambda i: (0, i, 0)),
            pl.BlockSpec((1, _R, H), lambda i: (1, i, 0)),
            pl.BlockSpec((D, H), lambda i: (0, 0)),
            pl.BlockSpec((1, H), lambda i: (0, 0)),
            pl.BlockSpec((1, H), lambda i: (0, 0)),
            pl.BlockSpec((D, H), lambda i: (0, 0)),
        ],
        out_specs=[
            pl.BlockSpec((_R, D), lambda i: (i, 0)),
            pl.BlockSpec((_R, H), lambda i: (i, 0)),
        ],
        out_shape=[
            jax.ShapeDtypeStruct((N, D), jnp.float32),
            jax.ShapeDtypeStruct((N, H), jnp.float32),
        ],
    )(h, parts, parts, W2, b2.reshape(1, H), b3.reshape(1, H), W3)


def _post_body(h_ref, p0_ref, p1_ref, w2_ref, b2_ref, b3_ref, w4_ref,
               b4_ref, w5_ref, b5_ref, out_ref):
    ln, rn = _combine(h_ref[...], p0_ref[0], p1_ref[0], w2_ref[...],
                      b2_ref[...], b3_ref[...])
    h4 = (jnp.dot(ln, w4_ref[:H, :], preferred_element_type=jnp.float32)
          + jnp.dot(rn, w4_ref[H:, :], preferred_element_type=jnp.float32)
          + b4_ref[...])
    h4 = jnp.maximum(h4, 0.0)
    out_ref[...] = (jnp.sum(h4 * w5_ref[...], axis=1, keepdims=True)
                    + b5_ref[...])


def _tc_post(h, parts, W2, b2, b3, W4, b4, W5, b5):
    return pl.pallas_call(
        _post_body,
        grid=(_GRID,),
        in_specs=[
            pl.BlockSpec((_R, D), lambda i: (i, 0)),
            pl.BlockSpec((1, _R, H), lambda i: (0, i, 0)),
            pl.BlockSpec((1, _R, H), lambda i: (1, i, 0)),
            pl.BlockSpec((D, H), lambda i: (0, 0)),
            pl.BlockSpec((1, H), lambda i: (0, 0)),
            pl.BlockSpec((1, H), lambda i: (0, 0)),
            pl.BlockSpec((D, D), lambda i: (0, 0)),
            pl.BlockSpec((1, D), lambda i: (0, 0)),
            pl.BlockSpec((1, D), lambda i: (0, 0)),
            pl.BlockSpec((1, 1), lambda i: (0, 0)),
        ],
        out_specs=pl.BlockSpec((_R, 1), lambda i: (i, 0)),
        out_shape=jax.ShapeDtypeStruct((N, 1), jnp.float32),
    )(h, parts, parts, W2, b2.reshape(1, H), b3.reshape(1, H), W4,
      b4.reshape(1, D), W5.reshape(1, D), b5.reshape(1, 1))


def kernel(x, edge_index, W1, b1, W2, b2, W3, b3, W4, b4, W5, b5):
    src = edge_index[0].astype(jnp.int32).reshape(_NW, _NSTEPS, _CHUNK)
    dst = edge_index[1].astype(jnp.int32).reshape(_NW, _NSTEPS, _CHUNK)
    zeros = jnp.zeros((N, H), jnp.float32)

    h, g = _tc_pre(x, W1, b1, W3)
    parts = _sc_segsum(g, src, dst, zeros)
    h, g = _tc_mid(h, parts, W2, b2, b3, W3)
    parts = _sc_segsum(g, src, dst, zeros)
    return _tc_post(h, parts, W2, b2, b3, W4, b4, W5, b5)


# speedup vs baseline: 11.3971x; 1.1552x over previous
import functools

import jax
import jax.numpy as jnp
from jax import lax
from jax.experimental import pallas as pl
from jax.experimental.pallas import tpu as pltpu
from jax.experimental.pallas import tpu_sc as plsc

N = 10000
E = 320000
D = 128
H = 64

_NC = 2
_NS = 16
_NW = _NC * _NS
_EPW = E // _NW
_CHUNK = 125
_NSTEPS = _EPW // _CHUNK
_RPT = 624
_TAIL = N - _RPT * _NS

_R = 2000
_GRID = N // _R


def _sc_body(g_hbm, src_hbm, dst_hbm, zero_hbm, out_hbm, sidx, didx, rows,
             acc, sem):
    cid = lax.axis_index("c")
    sid = lax.axis_index("s")
    wid = cid * _NS + sid

    s_cp = pltpu.async_copy(src_hbm.at[wid], sidx, sem)
    d_cp = pltpu.async_copy(dst_hbm.at[wid], didx, sem)

    pltpu.sync_copy(zero_hbm.at[pl.ds(sid * _RPT, _RPT)],
                    acc.at[pl.ds(sid * _RPT, _RPT)])

    @pl.when(sid == _NS - 1)
    def _():
        pltpu.sync_copy(zero_hbm.at[pl.ds(_RPT * _NS, _TAIL)],
                        acc.at[pl.ds(_RPT * _NS, _TAIL)])

    s_cp.wait()
    d_cp.wait()
    plsc.subcore_barrier()

    def gather(i, b):
        return pltpu.async_copy(g_hbm.at[sidx.at[i]], rows.at[b], sem)

    gather(0, 0).wait()

    def step(i, carry):
        b = lax.rem(i, 2)
        nxt = gather(lax.rem(i + 1, _NSTEPS), 1 - b)
        pltpu.sync_copy(rows.at[b], acc.at[didx.at[i]], add=True)
        nxt.wait()
        return carry

    lax.fori_loop(0, _NSTEPS, step, 0)

    plsc.subcore_barrier()
    pltpu.sync_copy(acc.at[pl.ds(sid * _RPT, _RPT)],
                    out_hbm.at[cid, pl.ds(sid * _RPT, _RPT)])

    @pl.when(sid == _NS - 1)
    def _():
        pltpu.sync_copy(acc.at[pl.ds(_RPT * _NS, _TAIL)],
                        out_hbm.at[cid, pl.ds(_RPT * _NS, _TAIL)])


@functools.cache
def _get_sc_segsum():
    return functools.partial(
        pl.kernel,
        out_type=jax.ShapeDtypeStruct((_NC, N, H), jnp.float32),
        mesh=plsc.VectorSubcoreMesh(core_axis_name="c", subcore_axis_name="s",
                                    num_cores=_NC, num_subcores=_NS),
        scratch_types=[
            pltpu.VMEM((_NSTEPS, _CHUNK), jnp.int32),
            pltpu.VMEM((_NSTEPS, _CHUNK), jnp.int32),
            pltpu.VMEM((2, _CHUNK, H), jnp.float32),
            pltpu.VMEM_SHARED((N, H), jnp.float32),
            pltpu.SemaphoreType.DMA,
        ],
        compiler_params=pltpu.CompilerParams(use_tc_tiling_on_sc=False),
    )(_sc_body)


def _sc_segsum(g, src, dst, zeros):
    return _get_sc_segsum()(g, src, dst, zeros)


def _pre_body(x_ref, w1_ref, b1_ref, w3_ref, h_ref, g_ref):
    h = jnp.dot(x_ref[...], w1_ref[...], preferred_element_type=jnp.float32)
    h = jnp.maximum(h + b1_ref[...], 0.0)
    hn = h * lax.rsqrt(jnp.sum(h * h, axis=1, keepdims=True))
    h_ref[...] = hn
    g_ref[...] = jnp.dot(hn, w3_ref[...], preferred_element_type=jnp.float32)


def _tc_pre(x, W1, b1, W3):
    return pl.pallas_call(
        _pre_body,
        grid=(_GRID,),
        in_specs=[
            pl.BlockSpec((_R, D), lambda i: (i, 0)),
            pl.BlockSpec((D, D), lambda i: (0, 0)),
            pl.BlockSpec((1, D), lambda i: (0, 0)),
            pl.BlockSpec((D, H), lambda i: (0, 0)),
        ],
        out_specs=[
            pl.BlockSpec((_R, D), lambda i: (i, 0)),
            pl.BlockSpec((_R, H), lambda i: (i, 0)),
        ],
        out_shape=[
            jax.ShapeDtypeStruct((N, D), jnp.float32),
            jax.ShapeDtypeStruct((N, H), jnp.float32),
        ],
    )(x, W1, b1.reshape(1, D), W3)


def _combine(h, p0, p1, w2, b2, b3):
    l = jnp.dot(h, w2, preferred_element_type=jnp.float32) + b2
    l = jnp.maximum(l, 0.0)
    r = jnp.maximum(p0 + p1 + b3, 0.0)
    ss = jnp.sum(l * l, axis=1, keepdims=True) + jnp.sum(r * r, axis=1,
                                                         keepdims=True)
    n = lax.rsqrt(ss)
    return l * n, r * n


def _mid_body(h_ref, p0_ref, p1_ref, w2_ref, b2_ref, b3_ref, w3_ref,
              h_out_ref, g_ref):
    ln, rn = _combine(h_ref[...], p0_ref[0], p1_ref[0], w2_ref[...],
                      b2_ref[...], b3_ref[...])
    h_out_ref[:, :H] = ln
    h_out_ref[:, H:] = rn
    g_ref[...] = (
        jnp.dot(ln, w3_ref[:H, :], preferred_element_type=jnp.float32)
        + jnp.dot(rn, w3_ref[H:, :], preferred_element_type=jnp.float32))


def _tc_mid(h, parts, W2, b2, b3, W3):
    return pl.pallas_call(
        _mid_body,
        grid=(_GRID,),
        in_specs=[
            pl.BlockSpec((_R, D), lambda i: (i, 0)),
            pl.BlockSpec((1, _R, H), lambda i: (0, i, 0)),
            pl.BlockSpec((1, _R, H), lambda i: (1, i, 0)),
            pl.BlockSpec((D, H), lambda i: (0, 0)),
            pl.BlockSpec((1, H), lambda i: (0, 0)),
            pl.BlockSpec((1, H), lambda i: (0, 0)),
            pl.BlockSpec((D, H), lambda i: (0, 0)),
        ],
        out_specs=[
            pl.BlockSpec((_R, D), lambda i: (i, 0)),
            pl.BlockSpec((_R, H), lambda i: (i, 0)),
        ],
        out_shape=[
            jax.ShapeDtypeStruct((N, D), jnp.float32),
            jax.ShapeDtypeStruct((N, H), jnp.float32),
        ],
    )(h, parts, parts, W2, b2.reshape(1, H), b3.reshape(1, H), W3)


def _post_body(h_ref, p0_ref, p1_ref, w2_ref, b2_ref, b3_ref, w4_ref,
               b4_ref, w5_ref, b5_ref, out_ref):
    ln, rn = _combine(h_ref[...], p0_ref[0], p1_ref[0], w2_ref[...],
                      b2_ref[...], b3_ref[...])
    h4 = (jnp.dot(ln, w4_ref[:H, :], preferred_element_type=jnp.float32)
          + jnp.dot(rn, w4_ref[H:, :], preferred_element_type=jnp.float32)
          + b4_ref[...])
    h4 = jnp.maximum(h4, 0.0)
    out_ref[...] = (jnp.sum(h4 * w5_ref[...], axis=1, keepdims=True)
                    + b5_ref[...])


def _tc_post(h, parts, W2, b2, b3, W4, b4, W5, b5):
    return pl.pallas_call(
        _post_body,
        grid=(_GRID,),
        in_specs=[
            pl.BlockSpec((_R, D), lambda i: (i, 0)),
            pl.BlockSpec((1, _R, H), lambda i: (0, i, 0)),
            pl.BlockSpec((1, _R, H), lambda i: (1, i, 0)),
            pl.BlockSpec((D, H), lambda i: (0, 0)),
            pl.BlockSpec((1, H), lambda i: (0, 0)),
            pl.BlockSpec((1, H), lambda i: (0, 0)),
            pl.BlockSpec((D, D), lambda i: (0, 0)),
            pl.BlockSpec((1, D), lambda i: (0, 0)),
            pl.BlockSpec((1, D), lambda i: (0, 0)),
            pl.BlockSpec((1, 1), lambda i: (0, 0)),
        ],
        out_specs=pl.BlockSpec((_R, 1), lambda i: (i, 0)),
        out_shape=jax.ShapeDtypeStruct((N, 1), jnp.float32),
    )(h, parts, parts, W2, b2.reshape(1, H), b3.reshape(1, H), W4,
      b4.reshape(1, D), W5.reshape(1, D), b5.reshape(1, 1))


def kernel(x, edge_index, W1, b1, W2, b2, W3, b3, W4, b4, W5, b5):
    src = edge_index[0].astype(jnp.int32).reshape(_NW, _NSTEPS, _CHUNK)
    dst = edge_index[1].astype(jnp.int32).reshape(_NW, _NSTEPS, _CHUNK)
    zeros = jnp.zeros((N, H), jnp.float32)

    h, g = _tc_pre(x, W1, b1, W3)
    parts = _sc_segsum(g, src, dst, zeros)
    h, g = _tc_mid(h, parts, W2, b2, b3, W3)
    parts = _sc_segsum(g, src, dst, zeros)
    return _tc_post(h, parts, W2, b2, b3, W4, b4, W5, b5)


# speedup vs baseline: 14.9586x; 1.3125x over previous
import functools

import jax
import jax.numpy as jnp
from jax import lax
from jax.experimental import pallas as pl
from jax.experimental.pallas import tpu as pltpu
from jax.experimental.pallas import tpu_sc as plsc

N = 10000
E = 320000
D = 128
H = 64

_NC = 2
_NS = 16
_NW = _NC * _NS
_EPW = E // _NW
_CHUNK = 125
_NSTEPS = _EPW // _CHUNK
_RPT = 624
_TAIL = N - _RPT * _NS

_R = 2000
_GRID = N // _R


def _sc_body(g_hbm, src_hbm, dst_hbm, zero_hbm, out_hbm, sidx, didx, rows,
             acc, sem, sem_s):
    cid = lax.axis_index("c")
    sid = lax.axis_index("s")
    wid = cid * _NS + sid

    s_cp = pltpu.async_copy(src_hbm.at[wid], sidx, sem)
    d_cp = pltpu.async_copy(dst_hbm.at[wid], didx, sem)

    pltpu.sync_copy(zero_hbm.at[pl.ds(sid * _RPT, _RPT)],
                    acc.at[pl.ds(sid * _RPT, _RPT)])

    @pl.when(sid == _NS - 1)
    def _():
        pltpu.sync_copy(zero_hbm.at[pl.ds(_RPT * _NS, _TAIL)],
                        acc.at[pl.ds(_RPT * _NS, _TAIL)])

    s_cp.wait()
    d_cp.wait()
    plsc.subcore_barrier()

    def gather(i, b):
        return pltpu.async_copy(g_hbm.at[sidx.at[i]], rows.at[b], sem)

    def gather_wait(i, b):
        pltpu.make_async_copy(g_hbm.at[sidx.at[i]], rows.at[b], sem).wait()

    def scat(i, b):
        return pltpu.async_copy(rows.at[b], acc.at[didx.at[i]], sem_s,
                                add=True)

    def scat_wait(i, b):
        pltpu.make_async_copy(rows.at[b], acc.at[didx.at[i]], sem_s).wait()

    gather(0, 0)
    gather(1, 1)
    gather_wait(0, 0)
    scat(0, 0)

    def step(i, carry):
        gather(lax.rem(i + 1, _NSTEPS), lax.rem(i + 1, 4))
        gather_wait(i, lax.rem(i, 4))
        scat(i, lax.rem(i, 4))
        scat_wait(i - 1, lax.rem(i - 1, 4))
        return carry

    lax.fori_loop(1, _NSTEPS, step, 0)
    gather_wait(0, lax.rem(_NSTEPS, 4))
    scat_wait(_NSTEPS - 1, lax.rem(_NSTEPS - 1, 4))

    plsc.subcore_barrier()
    pltpu.sync_copy(acc.at[pl.ds(sid * _RPT, _RPT)],
                    out_hbm.at[cid, pl.ds(sid * _RPT, _RPT)])

    @pl.when(sid == _NS - 1)
    def _():
        pltpu.sync_copy(acc.at[pl.ds(_RPT * _NS, _TAIL)],
                        out_hbm.at[cid, pl.ds(_RPT * _NS, _TAIL)])


@functools.cache
def _get_sc_segsum():
    return functools.partial(
        pl.kernel,
        out_type=jax.ShapeDtypeStruct((_NC, N, H), jnp.float32),
        mesh=plsc.VectorSubcoreMesh(core_axis_name="c", subcore_axis_name="s",
                                    num_cores=_NC, num_subcores=_NS),
        scratch_types=[
            pltpu.VMEM((_NSTEPS, _CHUNK), jnp.int32),
            pltpu.VMEM((_NSTEPS, _CHUNK), jnp.int32),
            pltpu.VMEM((4, _CHUNK, H), jnp.float32),
            pltpu.VMEM_SHARED((N, H), jnp.float32),
            pltpu.SemaphoreType.DMA,
            pltpu.SemaphoreType.DMA,
        ],
        compiler_params=pltpu.CompilerParams(use_tc_tiling_on_sc=False),
    )(_sc_body)


def _sc_segsum(g, src, dst, zeros):
    return _get_sc_segsum()(g, src, dst, zeros)


def _pre_body(x_ref, w1_ref, b1_ref, w3_ref, h_ref, g_ref):
    h = jnp.dot(x_ref[...], w1_ref[...], preferred_element_type=jnp.float32)
    h = jnp.maximum(h + b1_ref[...], 0.0)
    hn = h * lax.rsqrt(jnp.sum(h * h, axis=1, keepdims=True))
    h_ref[...] = hn
    g_ref[...] = jnp.dot(hn, w3_ref[...], preferred_element_type=jnp.float32)


def _tc_pre(x, W1, b1, W3):
    return pl.pallas_call(
        _pre_body,
        grid=(_GRID,),
        in_specs=[
            pl.BlockSpec((_R, D), lambda i: (i, 0)),
            pl.BlockSpec((D, D), lambda i: (0, 0)),
            pl.BlockSpec((1, D), lambda i: (0, 0)),
            pl.BlockSpec((D, H), lambda i: (0, 0)),
        ],
        out_specs=[
            pl.BlockSpec((_R, D), lambda i: (i, 0)),
            pl.BlockSpec((_R, H), lambda i: (i, 0)),
        ],
        out_shape=[
            jax.ShapeDtypeStruct((N, D), jnp.float32),
            jax.ShapeDtypeStruct((N, H), jnp.float32),
        ],
    )(x, W1, b1.reshape(1, D), W3)


def _combine(h, p0, p1, w2, b2, b3):
    l = jnp.dot(h, w2, preferred_element_type=jnp.float32) + b2
    l = jnp.maximum(l, 0.0)
    r = jnp.maximum(p0 + p1 + b3, 0.0)
    ss = jnp.sum(l * l, axis=1, keepdims=True) + jnp.sum(r * r, axis=1,
                                                         keepdims=True)
    n = lax.rsqrt(ss)
    return l * n, r * n


def _mid_body(h_ref, p0_ref, p1_ref, w2_ref, b2_ref, b3_ref, w3_ref,
              h_out_ref, g_ref):
    ln, rn = _combine(h_ref[...], p0_ref[0], p1_ref[0], w2_ref[...],
                      b2_ref[...], b3_ref[...])
    h_out_ref[:, :H] = ln
    h_out_ref[:, H:] = rn
    g_ref[...] = (
        jnp.dot(ln, w3_ref[:H, :], preferred_element_type=jnp.float32)
        + jnp.dot(rn, w3_ref[H:, :], preferred_element_type=jnp.float32))


def _tc_mid(h, parts, W2, b2, b3, W3):
    return pl.pallas_call(
        _mid_body,
        grid=(_GRID,),
        in_specs=[
            pl.BlockSpec((_R, D), lambda i: (i, 0)),
            pl.BlockSpec((1, _R, H), lambda i: (0, i, 0)),
            pl.BlockSpec((1, _R, H), lambda i: (1, i, 0)),
            pl.BlockSpec((D, H), lambda i: (0, 0)),
            pl.BlockSpec((1, H), lambda i: (0, 0)),
            pl.BlockSpec((1, H), lambda i: (0, 0)),
            pl.BlockSpec((D, H), lambda i: (0, 0)),
        ],
        out_specs=[
            pl.BlockSpec((_R, D), lambda i: (i, 0)),
            pl.BlockSpec((_R, H), lambda i: (i, 0)),
        ],
        out_shape=[
            jax.ShapeDtypeStruct((N, D), jnp.float32),
            jax.ShapeDtypeStruct((N, H), jnp.float32),
        ],
    )(h, parts, parts, W2, b2.reshape(1, H), b3.reshape(1, H), W3)


def _post_body(h_ref, p0_ref, p1_ref, w2_ref, b2_ref, b3_ref, w4_ref,
               b4_ref, w5_ref, b5_ref, out_ref):
    ln, rn = _combine(h_ref[...], p0_ref[0], p1_ref[0], w2_ref[...],
                      b2_ref[...], b3_ref[...])
    h4 = (jnp.dot(ln, w4_ref[:H, :], preferred_element_type=jnp.float32)
          + jnp.dot(rn, w4_ref[H:, :], preferred_element_type=jnp.float32)
          + b4_ref[...])
    h4 = jnp.maximum(h4, 0.0)
    out_ref[...] = (jnp.sum(h4 * w5_ref[...], axis=1, keepdims=True)
                    + b5_ref[...])


def _tc_post(h, parts, W2, b2, b3, W4, b4, W5, b5):
    return pl.pallas_call(
        _post_body,
        grid=(_GRID,),
        in_specs=[
            pl.BlockSpec((_R, D), lambda i: (i, 0)),
            pl.BlockSpec((1, _R, H), lambda i: (0, i, 0)),
            pl.BlockSpec((1, _R, H), lambda i: (1, i, 0)),
            pl.BlockSpec((D, H), lambda i: (0, 0)),
            pl.BlockSpec((1, H), lambda i: (0, 0)),
            pl.BlockSpec((1, H), lambda i: (0, 0)),
            pl.BlockSpec((D, D), lambda i: (0, 0)),
            pl.BlockSpec((1, D), lambda i: (0, 0)),
            pl.BlockSpec((1, D), lambda i: (0, 0)),
            pl.BlockSpec((1, 1), lambda i: (0, 0)),
        ],
        out_specs=pl.BlockSpec((_R, 1), lambda i: (i, 0)),
        out_shape=jax.ShapeDtypeStruct((N, 1), jnp.float32),
    )(h, parts, parts, W2, b2.reshape(1, H), b3.reshape(1, H), W4,
      b4.reshape(1, D), W5.reshape(1, D), b5.reshape(1, 1))


def kernel(x, edge_index, W1, b1, W2, b2, W3, b3, W4, b4, W5, b5):
    src = edge_index[0].astype(jnp.int32).reshape(_NW, _NSTEPS, _CHUNK)
    dst = edge_index[1].astype(jnp.int32).reshape(_NW, _NSTEPS, _CHUNK)
    zeros = jnp.zeros((N, H), jnp.float32)

    h, g = _tc_pre(x, W1, b1, W3)
    parts = _sc_segsum(g, src, dst, zeros)
    h, g = _tc_mid(h, parts, W2, b2, b3, W3)
    parts = _sc_segsum(g, src, dst, zeros)
    return _tc_post(h, parts, W2, b2, b3, W4, b4, W5, b5)


# speedup vs baseline: 16.5068x; 1.1035x over previous
import functools

import jax
import jax.numpy as jnp
from jax import lax
from jax.experimental import pallas as pl
from jax.experimental.pallas import tpu as pltpu
from jax.experimental.pallas import tpu_sc as plsc

N = 10000
E = 320000
D = 128
H = 64

_NC = 2
_NS = 16
_NW = _NC * _NS
_EPW = E // _NW
_CHUNK = 125
_NSTEPS = _EPW // _CHUNK
_RPT = 624
_TAIL = N - _RPT * _NS

_R = 2000
_GRID = N // _R


def _sc_body(g_hbm, src_hbm, dst_hbm, zero_hbm, out_hbm, sidx, didx, rows,
             acc, sem, sem_s):
    cid = lax.axis_index("c")
    sid = lax.axis_index("s")
    wid = cid * _NS + sid

    s_cp = pltpu.async_copy(src_hbm.at[wid], sidx, sem)
    d_cp = pltpu.async_copy(dst_hbm.at[wid], didx, sem)

    pltpu.sync_copy(zero_hbm.at[pl.ds(sid * _RPT, _RPT)],
                    acc.at[pl.ds(sid * _RPT, _RPT)])

    @pl.when(sid == _NS - 1)
    def _():
        pltpu.sync_copy(zero_hbm.at[pl.ds(_RPT * _NS, _TAIL)],
                        acc.at[pl.ds(_RPT * _NS, _TAIL)])

    s_cp.wait()
    d_cp.wait()
    plsc.subcore_barrier()

    def gather(i, b):
        return pltpu.async_copy(g_hbm.at[sidx.at[i]], rows.at[b], sem)

    def gather_wait(i, b):
        pltpu.make_async_copy(g_hbm.at[sidx.at[i]], rows.at[b], sem).wait()

    def scat(i, b):
        return pltpu.async_copy(rows.at[b], acc.at[didx.at[i]], sem_s,
                                add=True)

    def scat_wait(i, b):
        pltpu.make_async_copy(rows.at[b], acc.at[didx.at[i]], sem_s).wait()

    gather(0, 0)
    gather(1, 1)
    gather_wait(0, 0)
    scat(0, 0)

    def step(i, carry):
        gather(lax.rem(i + 1, _NSTEPS), lax.rem(i + 1, 4))
        gather_wait(i, lax.rem(i, 4))
        scat(i, lax.rem(i, 4))
        scat_wait(i - 1, lax.rem(i - 1, 4))
        return carry

    lax.fori_loop(1, _NSTEPS, step, 0)
    gather_wait(0, lax.rem(_NSTEPS, 4))
    scat_wait(_NSTEPS - 1, lax.rem(_NSTEPS - 1, 4))

    plsc.subcore_barrier()
    pltpu.sync_copy(acc.at[pl.ds(sid * _RPT, _RPT)],
                    out_hbm.at[pl.ds(sid * _RPT, _RPT), pl.ds(cid * H, H)])

    @pl.when(sid == _NS - 1)
    def _():
        pltpu.sync_copy(
            acc.at[pl.ds(_RPT * _NS, _TAIL)],
            out_hbm.at[pl.ds(_RPT * _NS, _TAIL), pl.ds(cid * H, H)])


@functools.cache
def _get_sc_segsum():
    return functools.partial(
        pl.kernel,
        out_type=jax.ShapeDtypeStruct((N, D), jnp.float32),
        mesh=plsc.VectorSubcoreMesh(core_axis_name="c", subcore_axis_name="s",
                                    num_cores=_NC, num_subcores=_NS),
        scratch_types=[
            pltpu.VMEM((_NSTEPS, _CHUNK), jnp.int32),
            pltpu.VMEM((_NSTEPS, _CHUNK), jnp.int32),
            pltpu.VMEM((4, _CHUNK, H), jnp.float32),
            pltpu.VMEM_SHARED((N, H), jnp.float32),
            pltpu.SemaphoreType.DMA,
            pltpu.SemaphoreType.DMA,
        ],
        compiler_params=pltpu.CompilerParams(use_tc_tiling_on_sc=False),
    )(_sc_body)


def _sc_segsum(g, src, dst, zeros):
    return _get_sc_segsum()(g, src, dst, zeros)


def _pre_body(x_ref, w1_ref, b1_ref, w3_ref, h_ref, g_ref):
    h = jnp.dot(x_ref[...], w1_ref[...], preferred_element_type=jnp.float32)
    h = jnp.maximum(h + b1_ref[...], 0.0)
    hn = h * lax.rsqrt(jnp.sum(h * h, axis=1, keepdims=True))
    h_ref[...] = hn
    g_ref[...] = jnp.dot(hn, w3_ref[...], preferred_element_type=jnp.float32)


def _tc_pre(x, W1, b1, W3):
    return pl.pallas_call(
        _pre_body,
        grid=(_GRID,),
        in_specs=[
            pl.BlockSpec((_R, D), lambda i: (i, 0)),
            pl.BlockSpec((D, D), lambda i: (0, 0)),
            pl.BlockSpec((1, D), lambda i: (0, 0)),
            pl.BlockSpec((D, H), lambda i: (0, 0)),
        ],
        out_specs=[
            pl.BlockSpec((_R, D), lambda i: (i, 0)),
            pl.BlockSpec((_R, H), lambda i: (i, 0)),
        ],
        out_shape=[
            jax.ShapeDtypeStruct((N, D), jnp.float32),
            jax.ShapeDtypeStruct((N, H), jnp.float32),
        ],
    )(x, W1, b1.reshape(1, D), W3)


def _combine(h, m, w2, b2, b3):
    l = jnp.dot(h, w2, preferred_element_type=jnp.float32) + b2
    l = jnp.maximum(l, 0.0)
    r = jnp.maximum(m[:, :H] + m[:, H:] + b3, 0.0)
    ss = jnp.sum(l * l, axis=1, keepdims=True) + jnp.sum(r * r, axis=1,
                                                         keepdims=True)
    n = lax.rsqrt(ss)
    return l * n, r * n


def _mid_body(h_ref, m_ref, w2_ref, b2_ref, b3_ref, w3_ref,
              h_out_ref, g_ref):
    ln, rn = _combine(h_ref[...], m_ref[...], w2_ref[...],
                      b2_ref[...], b3_ref[...])
    h_out_ref[:, :H] = ln
    h_out_ref[:, H:] = rn
    g_ref[...] = (
        jnp.dot(ln, w3_ref[:H, :], preferred_element_type=jnp.float32)
        + jnp.dot(rn, w3_ref[H:, :], preferred_element_type=jnp.float32))


def _tc_mid(h, msgs, W2, b2, b3, W3):
    return pl.pallas_call(
        _mid_body,
        grid=(_GRID,),
        in_specs=[
            pl.BlockSpec((_R, D), lambda i: (i, 0)),
            pl.BlockSpec((_R, D), lambda i: (i, 0)),
            pl.BlockSpec((D, H), lambda i: (0, 0)),
            pl.BlockSpec((1, H), lambda i: (0, 0)),
            pl.BlockSpec((1, H), lambda i: (0, 0)),
            pl.BlockSpec((D, H), lambda i: (0, 0)),
        ],
        out_specs=[
            pl.BlockSpec((_R, D), lambda i: (i, 0)),
            pl.BlockSpec((_R, H), lambda i: (i, 0)),
        ],
        out_shape=[
            jax.ShapeDtypeStruct((N, D), jnp.float32),
            jax.ShapeDtypeStruct((N, H), jnp.float32),
        ],
    )(h, msgs, W2, b2.reshape(1, H), b3.reshape(1, H), W3)


def _post_body(h_ref, m_ref, w2_ref, b2_ref, b3_ref, w4_ref,
               b4_ref, w5_ref, b5_ref, out_ref):
    ln, rn = _combine(h_ref[...], m_ref[...], w2_ref[...],
                      b2_ref[...], b3_ref[...])
    h4 = (jnp.dot(ln, w4_ref[:H, :], preferred_element_type=jnp.float32)
          + jnp.dot(rn, w4_ref[H:, :], preferred_element_type=jnp.float32)
          + b4_ref[...])
    h4 = jnp.maximum(h4, 0.0)
    out_ref[...] = (jnp.sum(h4 * w5_ref[...], axis=1, keepdims=True)
                    + b5_ref[...])


def _tc_post(h, msgs, W2, b2, b3, W4, b4, W5, b5):
    return pl.pallas_call(
        _post_body,
        grid=(_GRID,),
        in_specs=[
            pl.BlockSpec((_R, D), lambda i: (i, 0)),
            pl.BlockSpec((_R, D), lambda i: (i, 0)),
            pl.BlockSpec((D, H), lambda i: (0, 0)),
            pl.BlockSpec((1, H), lambda i: (0, 0)),
            pl.BlockSpec((1, H), lambda i: (0, 0)),
            pl.BlockSpec((D, D), lambda i: (0, 0)),
            pl.BlockSpec((1, D), lambda i: (0, 0)),
            pl.BlockSpec((1, D), lambda i: (0, 0)),
            pl.BlockSpec((1, 1), lambda i: (0, 0)),
        ],
        out_specs=pl.BlockSpec((_R, 1), lambda i: (i, 0)),
        out_shape=jax.ShapeDtypeStruct((N, 1), jnp.float32),
    )(h, msgs, W2, b2.reshape(1, H), b3.reshape(1, H), W4,
      b4.reshape(1, D), W5.reshape(1, D), b5.reshape(1, 1))


def kernel(x, edge_index, W1, b1, W2, b2, W3, b3, W4, b4, W5, b5):
    src = edge_index[0].astype(jnp.int32).reshape(_NW, _NSTEPS, _CHUNK)
    dst = edge_index[1].astype(jnp.int32).reshape(_NW, _NSTEPS, _CHUNK)
    zeros = jnp.zeros((N, H), jnp.float32)

    h, g = _tc_pre(x, W1, b1, W3)
    msgs = _sc_segsum(g, src, dst, zeros)
    h, g = _tc_mid(h, msgs, W2, b2, b3, W3)
    msgs = _sc_segsum(g, src, dst, zeros)
    return _tc_post(h, msgs, W2, b2, b3, W4, b4, W5, b5)
